# baseline (speedup 1.0000x reference)
import functools

import jax
import jax.numpy as jnp
from jax import lax
from jax.experimental import pallas as pl
from jax.experimental.pallas import tpu as pltpu
from jax.experimental.pallas import tpu_sc as plsc

N = 10000
NP = 10240
E = 320000
D8 = 8
DW = 16
NTILES = 32
CHUNK = 128
NCHUNK = 80
EPT = CHUNK * NCHUNK
EP = EPT * NTILES
ROWS_PT = NP // 16

_mesh = plsc.VectorSubcoreMesh(core_axis_name="c", subcore_axis_name="s")


@functools.partial(
    pl.kernel,
    mesh=_mesh,
    compiler_params=pltpu.CompilerParams(use_tc_tiling_on_sc=False),
    out_type=[jax.ShapeDtypeStruct((2, NP, DW), jnp.float32),
              jax.ShapeDtypeStruct((NP,), jnp.float32)],
    scratch_types=[
        pltpu.VMEM((2 * NCHUNK, CHUNK), jnp.int32),
        pltpu.VMEM((2 * NCHUNK, CHUNK), jnp.float32),
        pltpu.VMEM((NCHUNK, CHUNK), jnp.int32),
        pltpu.VMEM((NCHUNK, CHUNK), jnp.int32),
        pltpu.VMEM((NCHUNK, CHUNK), jnp.float32),
        pltpu.VMEM((ROWS_PT,), jnp.float32),
        pltpu.VMEM((ROWS_PT,), jnp.float32),
        pltpu.VMEM((CHUNK, DW), jnp.float32),
        pltpu.VMEM((CHUNK, DW), jnp.float32),
        pltpu.VMEM((CHUNK,), jnp.float32),
        pltpu.VMEM((CHUNK,), jnp.float32),
        pltpu.VMEM((CHUNK, DW), jnp.float32),
        pltpu.VMEM((CHUNK, DW), jnp.float32),
        pltpu.VMEM_SHARED((NP,), jnp.float32),
        pltpu.VMEM_SHARED((NP, DW), jnp.float32),
        pltpu.SemaphoreType.DMA,
        pltpu.SemaphoreType.DMA,
        pltpu.SemaphoreType.DMA,
        pltpu.SemaphoreType.DMA,
        pltpu.SemaphoreType.DMA,
        pltpu.SemaphoreType.DMA,
        pltpu.SemaphoreType.DMA,
    ],
)
def _sc_fused(row_hbm, col_hbm, ew_hbm, xw_hbm, z1_hbm, z8_hbm,
              s_hbm, dinv_hbm,
              colv2, ewv2, rowv, colv, ewv, dv, dinvv,
              rbuf0, rbuf1, dbuf0, dbuf1, mbuf0, mbuf1, dacc, sacc,
              dsem, gsem0, gsem1, hsem0, hsem1, ssem0, ssem1):
    c = lax.axis_index("c")
    s = lax.axis_index("s")
    t = c * 16 + s
    nslice = pl.ds(s * ROWS_PT, ROWS_PT)

    pltpu.sync_copy(z1_hbm.at[nslice], dacc.at[nslice])
    pltpu.sync_copy(z8_hbm.at[nslice], sacc.at[nslice])
    pltpu.sync_copy(col_hbm.at[2 * s], colv2.at[pl.ds(0, NCHUNK)])
    pltpu.sync_copy(col_hbm.at[2 * s + 1], colv2.at[pl.ds(NCHUNK, NCHUNK)])
    pltpu.sync_copy(ew_hbm.at[2 * s], ewv2.at[pl.ds(0, NCHUNK)])
    pltpu.sync_copy(ew_hbm.at[2 * s + 1], ewv2.at[pl.ds(NCHUNK, NCHUNK)])
    pltpu.sync_copy(row_hbm.at[t], rowv)
    pltpu.sync_copy(col_hbm.at[t], colv)
    pltpu.sync_copy(ew_hbm.at[t], ewv)
    plsc.subcore_barrier()

    def dbody(j, _):
        pltpu.async_copy(ewv2.at[j], dacc.at[colv2.at[j]], dsem, add=True)
        return _

    def ddrain(j, _):
        pltpu.make_async_copy(ewv2.at[j], dacc.at[colv2.at[j]], dsem).wait()
        return _

    lax.fori_loop(0, 2 * NCHUNK, dbody, None)
    lax.fori_loop(0, 2 * NCHUNK, ddrain, None)
    plsc.subcore_barrier()

    pltpu.sync_copy(dacc.at[nslice], dv)
    for k in range(ROWS_PT // 16):
        d16 = dv[pl.ds(16 * k, 16)] + 2.0
        yi = jnp.int32(0x5F3759DF) - (lax.bitcast_convert_type(d16, jnp.int32) >> 1)
        y = lax.bitcast_convert_type(yi, jnp.float32)
        for _it in range(3):
            y = y * (1.5 - 0.5 * d16 * y * y)
        dinvv[pl.ds(16 * k, 16)] = y
    pltpu.sync_copy(dinvv, dinv_hbm.at[nslice])
    plsc.subcore_barrier()

    rbufs = (rbuf0, rbuf1)
    dbufs = (dbuf0, dbuf1)
    mbufs = (mbuf0, mbuf1)
    gsems = (gsem0, gsem1)
    hsems = (hsem0, hsem1)
    ssems = (ssem0, ssem1)
    pltpu.async_copy(xw_hbm.at[rowv.at[0]], rbuf0, gsem0)
    pltpu.async_copy(dinv_hbm.at[rowv.at[0]], dbuf0, hsem0)

    def body2(jj, _):
        for b in range(2):
            j = 2 * jj + b

            @pl.when(j + 1 < NCHUNK)
            def _issue_next():
                pltpu.async_copy(xw_hbm.at[rowv.at[j + 1]],
                                 rbufs[1 - b], gsems[1 - b])
                pltpu.async_copy(dinv_hbm.at[rowv.at[j + 1]],
                                 dbufs[1 - b], hsems[1 - b])

            pltpu.make_async_copy(xw_hbm.at[rowv.at[j]],
                                  rbufs[b], gsems[b]).wait()
            pltpu.make_async_copy(dinv_hbm.at[rowv.at[j]],
                                  dbufs[b], hsems[b]).wait()

            @pl.when(j >= 2)
            def _drain_prev():
                pltpu.make_async_copy(mbufs[b], sacc.at[colv.at[j]],
                                      ssems[b]).wait()

            for p in range(CHUNK // 16):
                ew16 = ewv[j, pl.ds(16 * p, 16)]
                w16 = ew16 * dbufs[b][pl.ds(16 * p, 16)]
                for q in range(16):
                    i = 16 * p + q
                    mbufs[b][i, :] = rbufs[b][i, :] * w16[q]
            pltpu.async_copy(mbufs[b], sacc.at[colv.at[j]], ssems[b],
                             add=True)
        return _

    lax.fori_loop(0, NCHUNK // 2, body2, None)
    pltpu.make_async_copy(mbuf0, sacc.at[colv.at[0]], ssem0).wait()
    pltpu.make_async_copy(mbuf1, sacc.at[colv.at[0]], ssem1).wait()
    plsc.subcore_barrier()
    pltpu.sync_copy(sacc.at[nslice], s_hbm.at[c, nslice])


BLK = 1280


def _tc_matmul(x_ref, w_ref, xw_ref):
    xw = jnp.dot(x_ref[...], w_ref[...], preferred_element_type=jnp.float32)
    xw_ref[...] = jnp.concatenate(
        [xw, jnp.zeros((BLK, DW - D8), jnp.float32)], axis=1)


def _mat4(g, a_ref, r0):
    acc = g[:, 0:1] * a_ref[0:1, :]
    for k in range(1, 4):
        acc = acc + g[:, k:k + 1] * a_ref[k:k + 1, :]
    return acc + r0


def _tc_epilogue(sp_ref, xw_ref, dinv_ref, az_ref, ah_ref, bz_ref, bh_ref,
                 wo_ref, out_ref):
    S = sp_ref[0, :, 0:D8] + sp_ref[1, :, 0:D8]
    dinv = dinv_ref[...]
    G = dinv * (S + 2.0 * dinv * xw_ref[:, 0:D8])
    Z = jax.nn.sigmoid(_mat4(G[:, 0:4], az_ref, bz_ref[...]))
    Ht = jnp.tanh(_mat4(G[:, 4:8], ah_ref, bh_ref[...]))
    Hr = jax.nn.relu((1.0 - Z) * Ht)
    out_ref[...] = jnp.sum(Hr * wo_ref[0:1, :], axis=1, keepdims=True)


def kernel(x, edge_index, edge_weight, W_z, b_z, Wl_z, bl_z, W_r, b_r,
           Wl_r, bl_r, W_h, b_h, Wl_h, bl_h, W_out, b_out):
    f32 = jnp.float32
    row = edge_index[0].astype(jnp.int32)
    col = edge_index[1].astype(jnp.int32)
    ew = edge_weight.astype(f32)

    pad = EP - E
    row3 = jnp.concatenate([row, jnp.zeros((pad,), jnp.int32)]).reshape(
        NTILES, NCHUNK, CHUNK)
    col3 = jnp.concatenate([col, jnp.zeros((pad,), jnp.int32)]).reshape(
        NTILES, NCHUNK, CHUNK)
    ew3 = jnp.concatenate([ew, jnp.zeros((pad,), f32)]).reshape(
        NTILES, NCHUNK, CHUNK)

    x_pad = jnp.concatenate([x.astype(f32), jnp.zeros((NP - N, 128), f32)])
    w_cat = jnp.concatenate([W_z, W_h], axis=1).astype(f32)

    z1 = jnp.zeros((NP,), f32)
    z8 = jnp.zeros((NP, DW), f32)

    nblk = NP // BLK
    xw = pl.pallas_call(
        _tc_matmul,
        grid=(nblk,),
        in_specs=[
            pl.BlockSpec((BLK, 128), lambda i: (i, 0)),
            pl.BlockSpec((128, D8), lambda i: (0, 0)),
        ],
        out_specs=pl.BlockSpec((BLK, DW), lambda i: (i, 0)),
        out_shape=jax.ShapeDtypeStruct((NP, DW), f32),
    )(x_pad, w_cat)

    s_p, dinv_p = _sc_fused(row3, col3, ew3, xw, z1, z8)

    az = Wl_z[0:4].astype(f32)
    ah = Wl_h[0:4].astype(f32)
    bz_row = (b_z @ az + bl_z).reshape(1, 4).astype(f32)
    bh_row = (b_h @ ah + bl_h).reshape(1, 4).astype(f32)
    wo_row = W_out.reshape(1, 4).astype(f32)

    out = pl.pallas_call(
        _tc_epilogue,
        grid=(nblk,),
        in_specs=[
            pl.BlockSpec((2, BLK, DW), lambda i: (0, i, 0)),
            pl.BlockSpec((BLK, DW), lambda i: (i, 0)),
            pl.BlockSpec((BLK, 1), lambda i: (i, 0)),
            pl.BlockSpec((4, 4), lambda i: (0, 0)),
            pl.BlockSpec((4, 4), lambda i: (0, 0)),
            pl.BlockSpec((1, 4), lambda i: (0, 0)),
            pl.BlockSpec((1, 4), lambda i: (0, 0)),
            pl.BlockSpec((1, 4), lambda i: (0, 0)),
        ],
        out_specs=pl.BlockSpec((BLK, 1), lambda i: (i, 0)),
        out_shape=jax.ShapeDtypeStruct((NP, 1), f32),
    )(s_p, xw, dinv_p.reshape(NP, 1), az, ah, bz_row, bh_row, wo_row)

    return out[:N] + b_out


# speedup vs baseline: 1.0790x; 1.0790x over previous
import functools

import jax
import jax.numpy as jnp
from jax import lax
from jax.experimental import pallas as pl
from jax.experimental.pallas import tpu as pltpu
from jax.experimental.pallas import tpu_sc as plsc

N = 10000
NP = 10240
E = 320000
D8 = 8
DW = 16
NTILES = 32
CHUNK = 128
NCHUNK = 80
EPT = CHUNK * NCHUNK
EP = EPT * NTILES
ROWS_PT = NP // 16

_mesh = plsc.VectorSubcoreMesh(core_axis_name="c", subcore_axis_name="s")


@functools.partial(
    pl.kernel,
    mesh=_mesh,
    compiler_params=pltpu.CompilerParams(use_tc_tiling_on_sc=False),
    out_type=jax.ShapeDtypeStruct((2, NP), jnp.float32),
    scratch_types=[
        pltpu.VMEM((NCHUNK, CHUNK), jnp.int32),
        pltpu.VMEM((NCHUNK, CHUNK), jnp.float32),
        pltpu.VMEM_SHARED((NP,), jnp.float32),
        pltpu.SemaphoreType.DMA,
    ],
)
def _sc_degree(col_hbm, ew_hbm, z1_hbm, out_hbm, colv, ewv, dacc, dsem):
    c = lax.axis_index("c")
    s = lax.axis_index("s")
    t = c * 16 + s
    pltpu.sync_copy(z1_hbm.at[pl.ds(s * ROWS_PT, ROWS_PT)],
                    dacc.at[pl.ds(s * ROWS_PT, ROWS_PT)])
    pltpu.sync_copy(col_hbm.at[t], colv)
    pltpu.sync_copy(ew_hbm.at[t], ewv)
    plsc.subcore_barrier()

    def body(j, _):
        pltpu.async_copy(ewv.at[j], dacc.at[colv.at[j]], dsem, add=True)
        return _

    def drain(j, _):
        pltpu.make_async_copy(ewv.at[j], dacc.at[colv.at[j]], dsem).wait()
        return _

    lax.fori_loop(0, NCHUNK, body, None)
    lax.fori_loop(0, NCHUNK, drain, None)
    plsc.subcore_barrier()
    pltpu.sync_copy(dacc.at[pl.ds(s * ROWS_PT, ROWS_PT)],
                    out_hbm.at[c, pl.ds(s * ROWS_PT, ROWS_PT)])


@functools.partial(
    pl.kernel,
    mesh=_mesh,
    compiler_params=pltpu.CompilerParams(use_tc_tiling_on_sc=False),
    out_type=jax.ShapeDtypeStruct((2, NP, DW), jnp.float32),
    scratch_types=[
        pltpu.VMEM((NCHUNK, CHUNK), jnp.int32),
        pltpu.VMEM((NCHUNK, CHUNK), jnp.int32),
        pltpu.VMEM((NCHUNK, CHUNK), jnp.float32),
        pltpu.VMEM((CHUNK, DW), jnp.float32),
        pltpu.VMEM((CHUNK, DW), jnp.float32),
        pltpu.VMEM((CHUNK, DW), jnp.float32),
        pltpu.VMEM((CHUNK, DW), jnp.float32),
        pltpu.VMEM((CHUNK, DW), jnp.float32),
        pltpu.VMEM((CHUNK, DW), jnp.float32),
        pltpu.VMEM_SHARED((NP, DW), jnp.float32),
        pltpu.SemaphoreType.DMA,
        pltpu.SemaphoreType.DMA,
        pltpu.SemaphoreType.DMA,
        pltpu.SemaphoreType.DMA,
        pltpu.SemaphoreType.DMA,
        pltpu.SemaphoreType.DMA,
    ],
)
def _sc_messages(row_hbm, col_hbm, ew_hbm, xs_hbm, z8_hbm, out_hbm,
                 rowv, colv, ewv, rbuf0, rbuf1, rbuf2, mbuf0, mbuf1, mbuf2,
                 sacc, gsem0, gsem1, gsem2, ssem0, ssem1, ssem2):
    c = lax.axis_index("c")
    s = lax.axis_index("s")
    t = c * 16 + s
    pltpu.sync_copy(z8_hbm.at[pl.ds(s * ROWS_PT, ROWS_PT)],
                    sacc.at[pl.ds(s * ROWS_PT, ROWS_PT)])
    pltpu.sync_copy(row_hbm.at[t], rowv)
    pltpu.sync_copy(col_hbm.at[t], colv)
    pltpu.sync_copy(ew_hbm.at[t], ewv)
    plsc.subcore_barrier()

    rbufs = (rbuf0, rbuf1, rbuf2)
    mbufs = (mbuf0, mbuf1, mbuf2)
    gsems = (gsem0, gsem1, gsem2)
    ssems = (ssem0, ssem1, ssem2)
    pltpu.async_copy(xs_hbm.at[rowv.at[0]], rbuf0, gsem0)
    pltpu.async_copy(xs_hbm.at[rowv.at[1]], rbuf1, gsem1)

    def body3(jj, _):
        for b in range(3):
            j = 3 * jj + b

            @pl.when(j + 2 < NCHUNK)
            def _issue_next():
                pltpu.async_copy(xs_hbm.at[rowv.at[j + 2]],
                                 rbufs[(b + 2) % 3], gsems[(b + 2) % 3])

            pltpu.make_async_copy(xs_hbm.at[rowv.at[j]],
                                  rbufs[b], gsems[b]).wait()

            @pl.when(j >= 3)
            def _drain_prev():
                pltpu.make_async_copy(mbufs[b], sacc.at[colv.at[j]],
                                      ssems[b]).wait()

            for p in range(CHUNK // 16):
                ew16 = ewv[j, pl.ds(16 * p, 16)]
                for q in range(16):
                    i = 16 * p + q
                    mbufs[b][i, :] = rbufs[b][i, :] * ew16[q]
            pltpu.async_copy(mbufs[b], sacc.at[colv.at[j]], ssems[b],
                             add=True)
        return _

    lax.fori_loop(0, NCHUNK // 3, body3, None)
    for j0 in (78, 79):
        b = j0 % 3
        pltpu.make_async_copy(xs_hbm.at[rowv.at[j0]],
                              rbufs[b], gsems[b]).wait()
        pltpu.make_async_copy(mbufs[b], sacc.at[colv.at[0]],
                              ssems[b]).wait()
        for p in range(CHUNK // 16):
            ew16 = ewv[j0, pl.ds(16 * p, 16)]
            for q in range(16):
                i = 16 * p + q
                mbufs[b][i, :] = rbufs[b][i, :] * ew16[q]
        pltpu.async_copy(mbufs[b], sacc.at[colv.at[j0]], ssems[b],
                         add=True)
    for j0 in (77, 78, 79):
        b = j0 % 3
        pltpu.make_async_copy(mbufs[b], sacc.at[colv.at[0]],
                              ssems[b]).wait()
    plsc.subcore_barrier()
    pltpu.sync_copy(sacc.at[pl.ds(s * ROWS_PT, ROWS_PT)],
                    out_hbm.at[c, pl.ds(s * ROWS_PT, ROWS_PT)])


BLK = 1280


def _tc_prep(x_ref, w_ref, dp_ref, xs_ref, dinv_ref):
    xw = jnp.dot(x_ref[...], w_ref[...], preferred_element_type=jnp.float32)
    deg = dp_ref[0, :] + dp_ref[1, :] + 2.0
    dinv = lax.rsqrt(deg)
    xs_ref[...] = jnp.concatenate(
        [xw * dinv[:, None], jnp.zeros((BLK, DW - D8), jnp.float32)], axis=1)
    dinv_ref[...] = dinv[:, None]


def _mat4(g, a_ref, r0):
    acc = g[:, 0:1] * a_ref[0:1, :]
    for k in range(1, 4):
        acc = acc + g[:, k:k + 1] * a_ref[k:k + 1, :]
    return acc + r0


def _tc_epilogue(sp_ref, xs_ref, dinv_ref, az_ref, ah_ref, bz_ref, bh_ref,
                 wo_ref, out_ref):
    S = sp_ref[0, :, 0:D8] + sp_ref[1, :, 0:D8]
    G = dinv_ref[...] * (S + 2.0 * xs_ref[:, 0:D8])
    Z = jax.nn.sigmoid(_mat4(G[:, 0:4], az_ref, bz_ref[...]))
    Ht = jnp.tanh(_mat4(G[:, 4:8], ah_ref, bh_ref[...]))
    Hr = jax.nn.relu((1.0 - Z) * Ht)
    out_ref[...] = jnp.sum(Hr * wo_ref[0:1, :], axis=1, keepdims=True)


def kernel(x, edge_index, edge_weight, W_z, b_z, Wl_z, bl_z, W_r, b_r,
           Wl_r, bl_r, W_h, b_h, Wl_h, bl_h, W_out, b_out):
    f32 = jnp.float32
    row = edge_index[0].astype(jnp.int32)
    col = edge_index[1].astype(jnp.int32)
    ew = edge_weight.astype(f32)

    pad = EP - E
    row3 = jnp.concatenate([row, jnp.zeros((pad,), jnp.int32)]).reshape(
        NTILES, NCHUNK, CHUNK)
    col3 = jnp.concatenate([col, jnp.zeros((pad,), jnp.int32)]).reshape(
        NTILES, NCHUNK, CHUNK)
    ew3 = jnp.concatenate([ew, jnp.zeros((pad,), f32)]).reshape(
        NTILES, NCHUNK, CHUNK)

    x_pad = jnp.concatenate([x.astype(f32), jnp.zeros((NP - N, 128), f32)])
    w_cat = jnp.concatenate([W_z, W_h], axis=1).astype(f32)

    z1 = jnp.zeros((NP,), f32)
    z8 = jnp.zeros((NP, DW), f32)

    deg_p = _sc_degree(col3, ew3, z1)

    nblk = NP // BLK
    xs, dinv = pl.pallas_call(
        _tc_prep,
        grid=(nblk,),
        in_specs=[
            pl.BlockSpec((BLK, 128), lambda i: (i, 0)),
            pl.BlockSpec((128, D8), lambda i: (0, 0)),
            pl.BlockSpec((2, BLK), lambda i: (0, i)),
        ],
        out_specs=[
            pl.BlockSpec((BLK, DW), lambda i: (i, 0)),
            pl.BlockSpec((BLK, 1), lambda i: (i, 0)),
        ],
        out_shape=[jax.ShapeDtypeStruct((NP, DW), f32),
                   jax.ShapeDtypeStruct((NP, 1), f32)],
    )(x_pad, w_cat, deg_p)

    s_p = _sc_messages(row3, col3, ew3, xs, z8)

    az = Wl_z[0:4].astype(f32)
    ah = Wl_h[0:4].astype(f32)
    bz_row = (b_z @ az + bl_z).reshape(1, 4).astype(f32)
    bh_row = (b_h @ ah + bl_h).reshape(1, 4).astype(f32)
    wo_row = W_out.reshape(1, 4).astype(f32)

    out = pl.pallas_call(
        _tc_epilogue,
        grid=(nblk,),
        in_specs=[
            pl.BlockSpec((2, BLK, DW), lambda i: (0, i, 0)),
            pl.BlockSpec((BLK, DW), lambda i: (i, 0)),
            pl.BlockSpec((BLK, 1), lambda i: (i, 0)),
            pl.BlockSpec((4, 4), lambda i: (0, 0)),
            pl.BlockSpec((4, 4), lambda i: (0, 0)),
            pl.BlockSpec((1, 4), lambda i: (0, 0)),
            pl.BlockSpec((1, 4), lambda i: (0, 0)),
            pl.BlockSpec((1, 4), lambda i: (0, 0)),
        ],
        out_specs=pl.BlockSpec((BLK, 1), lambda i: (i, 0)),
        out_shape=jax.ShapeDtypeStruct((NP, 1), f32),
    )(s_p, xs, dinv, az, ah, bz_row, bh_row, wo_row)

    return out[:N] + b_out


# speedup vs baseline: 1.2063x; 1.1180x over previous
import functools

import jax
import jax.numpy as jnp
from jax import lax
from jax.experimental import pallas as pl
from jax.experimental.pallas import tpu as pltpu
from jax.experimental.pallas import tpu_sc as plsc

N = 10000
NP = 10240
E = 320000
D8 = 8
DW = 16
NTILES = 32
CHUNK = 128
NCHUNK = 80
EPT = CHUNK * NCHUNK
EP = EPT * NTILES
ROWS_PT = NP // 16

_mesh = plsc.VectorSubcoreMesh(core_axis_name="c", subcore_axis_name="s")


@functools.partial(
    pl.kernel,
    mesh=_mesh,
    compiler_params=pltpu.CompilerParams(use_tc_tiling_on_sc=False),
    out_type=jax.ShapeDtypeStruct((2, NP), jnp.float32),
    scratch_types=[
        pltpu.VMEM((NCHUNK, CHUNK), jnp.int32),
        pltpu.VMEM((NCHUNK, CHUNK), jnp.float32),
        pltpu.VMEM_SHARED((NP,), jnp.float32),
        pltpu.SemaphoreType.DMA,
    ],
)
def _sc_degree(col_hbm, ew_hbm, z1_hbm, out_hbm, colv, ewv, dacc, dsem):
    c = lax.axis_index("c")
    s = lax.axis_index("s")
    t = c * 16 + s
    pltpu.sync_copy(z1_hbm.at[pl.ds(s * ROWS_PT, ROWS_PT)],
                    dacc.at[pl.ds(s * ROWS_PT, ROWS_PT)])
    pltpu.sync_copy(col_hbm.at[t], colv)
    pltpu.sync_copy(ew_hbm.at[t], ewv)
    plsc.subcore_barrier()

    def body(j, _):
        pltpu.async_copy(ewv.at[j], dacc.at[colv.at[j]], dsem, add=True)
        return _

    def drain(j, _):
        pltpu.make_async_copy(ewv.at[j], dacc.at[colv.at[j]], dsem).wait()
        return _

    lax.fori_loop(0, NCHUNK, body, None)
    lax.fori_loop(0, NCHUNK, drain, None)
    plsc.subcore_barrier()
    pltpu.sync_copy(dacc.at[pl.ds(s * ROWS_PT, ROWS_PT)],
                    out_hbm.at[c, pl.ds(s * ROWS_PT, ROWS_PT)])


@functools.partial(
    pl.kernel,
    mesh=_mesh,
    compiler_params=pltpu.CompilerParams(use_tc_tiling_on_sc=False),
    out_type=jax.ShapeDtypeStruct((2, NP, DW), jnp.float32),
    scratch_types=[
        pltpu.VMEM((NCHUNK, CHUNK), jnp.int32),
        pltpu.VMEM((NCHUNK, CHUNK), jnp.int32),
        pltpu.VMEM((NCHUNK, CHUNK), jnp.float32),
        pltpu.VMEM((CHUNK, DW), jnp.float32),
        pltpu.VMEM((CHUNK, DW), jnp.float32),
        pltpu.VMEM((CHUNK, DW), jnp.float32),
        pltpu.VMEM((CHUNK, DW), jnp.float32),
        pltpu.VMEM((CHUNK, DW), jnp.float32),
        pltpu.VMEM((CHUNK, DW), jnp.float32),
        pltpu.VMEM_SHARED((NP, DW), jnp.float32),
        pltpu.SemaphoreType.DMA,
        pltpu.SemaphoreType.DMA,
        pltpu.SemaphoreType.DMA,
        pltpu.SemaphoreType.DMA,
        pltpu.SemaphoreType.DMA,
        pltpu.SemaphoreType.DMA,
    ],
)
def _sc_messages(row_hbm, col_hbm, ew_hbm, xs_hbm, z8_hbm, out_hbm,
                 rowv, colv, ewv, rbuf0, rbuf1, rbuf2, mbuf0, mbuf1, mbuf2,
                 sacc, gsem0, gsem1, gsem2, ssem0, ssem1, ssem2):
    c = lax.axis_index("c")
    s = lax.axis_index("s")
    t = c * 16 + s
    pltpu.sync_copy(z8_hbm.at[pl.ds(s * ROWS_PT, ROWS_PT)],
                    sacc.at[pl.ds(s * ROWS_PT, ROWS_PT)])
    pltpu.sync_copy(row_hbm.at[t], rowv)
    pltpu.sync_copy(col_hbm.at[t], colv)
    pltpu.sync_copy(ew_hbm.at[t], ewv)
    plsc.subcore_barrier()

    rbufs = (rbuf0, rbuf1, rbuf2)
    mbufs = (mbuf0, mbuf1, mbuf2)
    gsems = (gsem0, gsem1, gsem2)
    ssems = (ssem0, ssem1, ssem2)
    pltpu.async_copy(xs_hbm.at[rowv.at[0]], rbuf0, gsem0)
    pltpu.async_copy(xs_hbm.at[rowv.at[1]], rbuf1, gsem1)

    def body3(jj, _):
        for b in range(3):
            j = 3 * jj + b

            @pl.when(j + 2 < NCHUNK)
            def _issue_next():
                pltpu.async_copy(xs_hbm.at[rowv.at[j + 2]],
                                 rbufs[(b + 2) % 3], gsems[(b + 2) % 3])

            pltpu.make_async_copy(xs_hbm.at[rowv.at[j]],
                                  rbufs[b], gsems[b]).wait()

            @pl.when(j >= 3)
            def _drain_prev():
                pltpu.make_async_copy(mbufs[b], sacc.at[colv.at[j]],
                                      ssems[b]).wait()

            for p in range(CHUNK // 16):
                ew16 = ewv[j, pl.ds(16 * p, 16)]
                for q in range(16):
                    i = 16 * p + q
                    mbufs[b][i, :] = rbufs[b][i, :] * ew16[q]
            pltpu.async_copy(mbufs[b], sacc.at[colv.at[j]], ssems[b],
                             add=True)
        return _

    lax.fori_loop(0, NCHUNK // 3, body3, None)
    for j0 in (78, 79):
        b = j0 % 3
        pltpu.make_async_copy(xs_hbm.at[rowv.at[j0]],
                              rbufs[b], gsems[b]).wait()
        pltpu.make_async_copy(mbufs[b], sacc.at[colv.at[0]],
                              ssems[b]).wait()
        for p in range(CHUNK // 16):
            ew16 = ewv[j0, pl.ds(16 * p, 16)]
            for q in range(16):
                i = 16 * p + q
                mbufs[b][i, :] = rbufs[b][i, :] * ew16[q]
        pltpu.async_copy(mbufs[b], sacc.at[colv.at[j0]], ssems[b],
                         add=True)
    for j0 in (77, 78, 79):
        b = j0 % 3
        pltpu.make_async_copy(mbufs[b], sacc.at[colv.at[0]],
                              ssems[b]).wait()
    plsc.subcore_barrier()
    pltpu.sync_copy(sacc.at[pl.ds(s * ROWS_PT, ROWS_PT)],
                    out_hbm.at[c, pl.ds(s * ROWS_PT, ROWS_PT)])


BLK = 1280


def _tc_matmul(x_ref, w_ref, xw_ref):
    xw_ref[...] = jnp.dot(x_ref[...], w_ref[...],
                          preferred_element_type=jnp.float32)


def _tc_scale(xw_ref, dp_ref, xs_ref, dinv_ref):
    deg = dp_ref[0, :] + dp_ref[1, :] + 2.0
    dinv = lax.rsqrt(deg)
    xs_ref[...] = jnp.concatenate(
        [xw_ref[...] * dinv[:, None], jnp.zeros((BLK, DW - D8), jnp.float32)],
        axis=1)
    dinv_ref[...] = dinv[:, None]


def _mat4(g, a_ref, r0):
    acc = g[:, 0:1] * a_ref[0:1, :]
    for k in range(1, 4):
        acc = acc + g[:, k:k + 1] * a_ref[k:k + 1, :]
    return acc + r0


def _tc_epilogue(sp_ref, xs_ref, dinv_ref, az_ref, ah_ref, bz_ref, bh_ref,
                 wo_ref, out_ref):
    S = sp_ref[0, :, 0:D8] + sp_ref[1, :, 0:D8]
    G = dinv_ref[...] * (S + 2.0 * xs_ref[:, 0:D8])
    Z = jax.nn.sigmoid(_mat4(G[:, 0:4], az_ref, bz_ref[...]))
    Ht = jnp.tanh(_mat4(G[:, 4:8], ah_ref, bh_ref[...]))
    Hr = jax.nn.relu((1.0 - Z) * Ht)
    out_ref[...] = jnp.sum(Hr * wo_ref[0:1, :], axis=1, keepdims=True)


def kernel(x, edge_index, edge_weight, W_z, b_z, Wl_z, bl_z, W_r, b_r,
           Wl_r, bl_r, W_h, b_h, Wl_h, bl_h, W_out, b_out):
    f32 = jnp.float32
    row = edge_index[0].astype(jnp.int32)
    col = edge_index[1].astype(jnp.int32)
    ew = edge_weight.astype(f32)

    pad = EP - E
    row3 = jnp.concatenate([row, jnp.zeros((pad,), jnp.int32)]).reshape(
        NTILES, NCHUNK, CHUNK)
    col3 = jnp.concatenate([col, jnp.zeros((pad,), jnp.int32)]).reshape(
        NTILES, NCHUNK, CHUNK)
    ew3 = jnp.concatenate([ew, jnp.zeros((pad,), f32)]).reshape(
        NTILES, NCHUNK, CHUNK)

    x_pad = jnp.concatenate([x.astype(f32), jnp.zeros((NP - N, 128), f32)])
    w_cat = jnp.concatenate([W_z, W_h], axis=1).astype(f32)

    z1 = jnp.zeros((NP,), f32)
    z8 = jnp.zeros((NP, DW), f32)

    nblk = NP // BLK
    xw = pl.pallas_call(
        _tc_matmul,
        grid=(nblk,),
        in_specs=[
            pl.BlockSpec((BLK, 128), lambda i: (i, 0)),
            pl.BlockSpec((128, D8), lambda i: (0, 0)),
        ],
        out_specs=pl.BlockSpec((BLK, D8), lambda i: (i, 0)),
        out_shape=jax.ShapeDtypeStruct((NP, D8), f32),
    )(x_pad, w_cat)

    deg_p = _sc_degree(col3, ew3, z1)

    xs, dinv = pl.pallas_call(
        _tc_scale,
        grid=(nblk,),
        in_specs=[
            pl.BlockSpec((BLK, D8), lambda i: (i, 0)),
            pl.BlockSpec((2, BLK), lambda i: (0, i)),
        ],
        out_specs=[
            pl.BlockSpec((BLK, DW), lambda i: (i, 0)),
            pl.BlockSpec((BLK, 1), lambda i: (i, 0)),
        ],
        out_shape=[jax.ShapeDtypeStruct((NP, DW), f32),
                   jax.ShapeDtypeStruct((NP, 1), f32)],
    )(xw, deg_p)

    s_p = _sc_messages(row3, col3, ew3, xs, z8)

    az = Wl_z[0:4].astype(f32)
    ah = Wl_h[0:4].astype(f32)
    bz_row = (b_z @ az + bl_z).reshape(1, 4).astype(f32)
    bh_row = (b_h @ ah + bl_h).reshape(1, 4).astype(f32)
    wo_row = W_out.reshape(1, 4).astype(f32)

    out = pl.pallas_call(
        _tc_epilogue,
        grid=(nblk,),
        in_specs=[
            pl.BlockSpec((2, BLK, DW), lambda i: (0, i, 0)),
            pl.BlockSpec((BLK, DW), lambda i: (i, 0)),
            pl.BlockSpec((BLK, 1), lambda i: (i, 0)),
            pl.BlockSpec((4, 4), lambda i: (0, 0)),
            pl.BlockSpec((4, 4), lambda i: (0, 0)),
            pl.BlockSpec((1, 4), lambda i: (0, 0)),
            pl.BlockSpec((1, 4), lambda i: (0, 0)),
            pl.BlockSpec((1, 4), lambda i: (0, 0)),
        ],
        out_specs=pl.BlockSpec((BLK, 1), lambda i: (i, 0)),
        out_shape=jax.ShapeDtypeStruct((NP, 1), f32),
    )(s_p, xs, dinv, az, ah, bz_row, bh_row, wo_row)

    return out[:N] + b_out
